# baseline (speedup 1.0000x reference)
import jax
import jax.numpy as jnp
from jax import lax
from jax.experimental import pallas as pl
from jax.experimental.pallas import tpu as pltpu
from jax.experimental.pallas import tpu_sc as plsc

_NC = 2
_NS = 16
_NW = _NC * _NS
_GW = 128
_DEPTH = 4


def kernel(x, E_absolute_position):
    B, H = x.shape
    N, D = E_absolute_position.shape
    num_indices = B * H
    per_w = num_indices // _NW
    n_chunks = per_w // _GW
    n_groups = n_chunks // _DEPTH
    assert per_w * _NW == num_indices
    assert n_groups * _DEPTH * _GW == per_w

    idx = x.reshape(_NW, n_chunks, _GW).astype(jnp.int32)

    mesh = plsc.VectorSubcoreMesh(core_axis_name="c", subcore_axis_name="s")

    scratch = (
        [pltpu.VMEM((n_chunks, _GW), jnp.int32)]
        + [pltpu.VMEM((_GW, D), jnp.float32) for _ in range(_DEPTH)]
        + [pltpu.SemaphoreType.DMA for _ in range(2 * _DEPTH)]
    )

    @pl.kernel(
        out_type=jax.ShapeDtypeStruct((num_indices, D),
                                      E_absolute_position.dtype),
        mesh=mesh,
        scratch_types=scratch,
        compiler_params=pltpu.CompilerParams(use_tc_tiling_on_sc=False),
    )
    def gather_kernel(table_hbm, idx_hbm, out_hbm, idx_v, *scr):
        rows = scr[:_DEPTH]
        gsem = scr[_DEPTH:2 * _DEPTH]
        wsem = scr[2 * _DEPTH:]
        wid = lax.axis_index("s") * _NC + lax.axis_index("c")
        base = wid * per_w

        pltpu.sync_copy(idx_hbm.at[wid], idx_v)

        def start_gather(c, k):
            pltpu.make_async_copy(table_hbm.at[idx_v.at[c]], rows[k],
                                  gsem[k]).start()

        def wait_gather(c, k):
            pltpu.make_async_copy(table_hbm.at[idx_v.at[c]], rows[k],
                                  gsem[k]).wait()

        def out_copy(c, k):
            return pltpu.make_async_copy(
                rows[k], out_hbm.at[pl.ds(base + c * _GW, _GW)], wsem[k])

        for k in range(_DEPTH):
            start_gather(k, k)

        @pl.loop(0, n_groups)
        def _(t):
            c0 = t * _DEPTH
            for k in range(_DEPTH):
                wait_gather(c0 + k, k)
                out_copy(c0 + k, k).start()
            for k in range(_DEPTH):
                cn = lax.rem(c0 + k + _DEPTH, n_chunks)
                out_copy(c0 + k, k).wait()
                start_gather(cn, k)

        for k in range(_DEPTH):
            wait_gather(k, k)

    out = gather_kernel(E_absolute_position, idx)
    return out.reshape(B, H, D)


# speedup vs baseline: 1.3287x; 1.3287x over previous
import jax
import jax.numpy as jnp
from jax import lax
from jax.experimental import pallas as pl
from jax.experimental.pallas import tpu as pltpu
from jax.experimental.pallas import tpu_sc as plsc

_NC = 2
_NS = 16
_NW = _NC * _NS
_GW = 128
_DEPTH = 4


def kernel(x, E_absolute_position):
    B, H = x.shape
    N, D = E_absolute_position.shape
    num_indices = B * H
    per_w = num_indices // _NW
    n_chunks = per_w // _GW
    n_groups = n_chunks // _DEPTH
    assert per_w * _NW == num_indices
    assert n_groups * _DEPTH * _GW == per_w

    idx = x.reshape(_NW, n_chunks, _GW).astype(jnp.int32)

    mesh = plsc.VectorSubcoreMesh(core_axis_name="c", subcore_axis_name="s")

    scratch = (
        [pltpu.VMEM((n_chunks, _GW), jnp.int32)]
        + [pltpu.VMEM((_GW, D), jnp.float32) for _ in range(_DEPTH)]
        + [pltpu.SemaphoreType.DMA for _ in range(2 * _DEPTH)]
    )

    @pl.kernel(
        out_type=jax.ShapeDtypeStruct((num_indices, 128),
                                      E_absolute_position.dtype),
        mesh=mesh,
        scratch_types=scratch,
        compiler_params=pltpu.CompilerParams(use_tc_tiling_on_sc=False),
    )
    def gather_kernel(table_hbm, idx_hbm, out_hbm, idx_v, *scr):
        rows = scr[:_DEPTH]
        gsem = scr[_DEPTH:2 * _DEPTH]
        wsem = scr[2 * _DEPTH:]
        wid = lax.axis_index("s") * _NC + lax.axis_index("c")
        base = wid * per_w

        pltpu.sync_copy(idx_hbm.at[wid], idx_v)

        def gth(c, k):
            return pltpu.make_async_copy(
                table_hbm.at[idx_v.at[c]], rows[k], gsem[k])

        def wrt(c, k):
            return pltpu.make_async_copy(
                rows[k],
                out_hbm.at[pl.ds(base + c * _GW, _GW), pl.ds(0, D)],
                wsem[k])

        for k in range(_DEPTH):
            gth(k, k).start()

        @pl.loop(0, n_groups)
        def _(t):
            c0 = t * _DEPTH
            for k in range(_DEPTH):
                gth(c0 + k, k).wait()
                wrt(c0 + k, k).start()
            for k in range(_DEPTH):
                cn = lax.rem(c0 + k + _DEPTH, n_chunks)
                wrt(c0 + k, k).wait()
                gth(cn, k).start()

        for k in range(_DEPTH):
            gth(k, k).wait()

    out = gather_kernel(E_absolute_position, idx)
    return out[:, :D].reshape(B, H, D)


# speedup vs baseline: 1.5611x; 1.1749x over previous
import jax
import jax.numpy as jnp
from jax import lax
from jax.experimental import pallas as pl
from jax.experimental.pallas import tpu as pltpu
from jax.experimental.pallas import tpu_sc as plsc

_NC = 2
_NS = 16
_NW = _NC * _NS
_GW = 128
_DEPTH = 4
_BC = 2048


def _repack_block(t_ref, o_ref):
    h = _BC // 2
    o_ref[:, :64] = t_ref[:, :h].T
    o_ref[:, 64:] = t_ref[:, h:].T


def _repack(table_t):
    D, N = table_t.shape
    n_blocks = (N + _BC - 1) // _BC
    return pl.pallas_call(
        _repack_block,
        grid=(n_blocks,),
        in_specs=[pl.BlockSpec((D, _BC), lambda i: (0, i))],
        out_specs=pl.BlockSpec((_BC // 2, 2 * D), lambda i: (i, 0)),
        out_shape=jax.ShapeDtypeStruct((n_blocks * (_BC // 2), 2 * D),
                                       table_t.dtype),
    )(table_t)


def kernel(x, E_absolute_position):
    B, H = x.shape
    N, D = E_absolute_position.shape
    num_indices = B * H
    per_w = num_indices // _NW
    n_chunks = per_w // _GW
    n_groups = n_chunks // _DEPTH
    assert per_w * _NW == num_indices
    assert n_groups * _DEPTH * _GW == per_w

    x32 = x.astype(jnp.int32)
    j = x32 & (_BC - 1)
    idx = 2 * x32 - (x32 - j) - jnp.where(j < _BC // 2, 0, _BC - 1)
    idx = idx.reshape(_NW, n_chunks, _GW)

    packed = _repack(E_absolute_position.T)
    table_lin = packed.reshape(packed.shape[0] * 2, D)

    mesh = plsc.VectorSubcoreMesh(core_axis_name="c", subcore_axis_name="s")

    scratch = (
        [pltpu.VMEM((n_chunks, _GW), jnp.int32)]
        + [pltpu.VMEM((_GW, D), jnp.float32) for _ in range(_DEPTH)]
        + [pltpu.SemaphoreType.DMA for _ in range(2 * _DEPTH)]
    )

    @pl.kernel(
        out_type=jax.ShapeDtypeStruct((num_indices, 128),
                                      E_absolute_position.dtype),
        mesh=mesh,
        scratch_types=scratch,
        compiler_params=pltpu.CompilerParams(use_tc_tiling_on_sc=False),
    )
    def gather_kernel(table_hbm, idx_hbm, out_hbm, idx_v, *scr):
        rows = scr[:_DEPTH]
        gsem = scr[_DEPTH:2 * _DEPTH]
        wsem = scr[2 * _DEPTH:]
        wid = lax.axis_index("s") * _NC + lax.axis_index("c")
        base = wid * per_w

        pltpu.sync_copy(idx_hbm.at[wid], idx_v)

        def gth(c, k):
            return pltpu.make_async_copy(
                table_hbm.at[idx_v.at[c]], rows[k], gsem[k])

        def wrt(c, k):
            return pltpu.make_async_copy(
                rows[k],
                out_hbm.at[pl.ds(base + c * _GW, _GW), pl.ds(0, D)],
                wsem[k])

        for k in range(_DEPTH):
            gth(k, k).start()

        @pl.loop(0, n_groups)
        def _(t):
            c0 = t * _DEPTH
            for k in range(_DEPTH):
                gth(c0 + k, k).wait()
                wrt(c0 + k, k).start()
            for k in range(_DEPTH):
                cn = lax.rem(c0 + k + _DEPTH, n_chunks)
                wrt(c0 + k, k).wait()
                gth(cn, k).start()

        for k in range(_DEPTH):
            gth(k, k).wait()

    out = gather_kernel(table_lin, idx)
    return out[:, :D].reshape(B, H, D)


# speedup vs baseline: 1.8407x; 1.1792x over previous
import jax
import jax.numpy as jnp
from jax import lax
from jax.experimental import pallas as pl
from jax.experimental.pallas import tpu as pltpu
from jax.experimental.pallas import tpu_sc as plsc

_NC = 2
_NS = 16
_NW = _NC * _NS
_GW = 128
_DEPTH = 4
_BC = 4096


def _repack_block(t_ref, o_ref):
    h = _BC // 2
    o_ref[:, :64] = t_ref[:, :h].T
    o_ref[:, 64:] = t_ref[:, h:].T


def _repack(table_t):
    D, N = table_t.shape
    n_blocks = (N + _BC - 1) // _BC
    return pl.pallas_call(
        _repack_block,
        grid=(n_blocks,),
        in_specs=[pl.BlockSpec((D, _BC), lambda i: (0, i))],
        out_specs=pl.BlockSpec((_BC // 2, 2 * D), lambda i: (i, 0)),
        out_shape=jax.ShapeDtypeStruct((n_blocks * (_BC // 2), 2 * D),
                                       table_t.dtype),
    )(table_t)


def kernel(x, E_absolute_position):
    B, H = x.shape
    N, D = E_absolute_position.shape
    num_indices = B * H
    per_w = num_indices // _NW
    n_chunks = per_w // _GW
    n_groups = n_chunks // _DEPTH
    assert per_w * _NW == num_indices
    assert n_groups * _DEPTH * _GW == per_w

    x32 = x.astype(jnp.int32)
    j = x32 & (_BC - 1)
    idx = 2 * x32 - (x32 - j) - jnp.where(j < _BC // 2, 0, _BC - 1)
    idx = idx.reshape(_NW, n_chunks, _GW)

    packed = _repack(E_absolute_position.T)
    table_lin = packed.reshape(packed.shape[0] * 2, D)

    mesh = plsc.VectorSubcoreMesh(core_axis_name="c", subcore_axis_name="s")

    scratch = (
        [pltpu.VMEM((n_chunks, _GW), jnp.int32)]
        + [pltpu.VMEM((_GW, D), jnp.float32) for _ in range(_DEPTH)]
        + [pltpu.SemaphoreType.DMA for _ in range(2 * _DEPTH)]
    )

    @pl.kernel(
        out_type=jax.ShapeDtypeStruct((num_indices, 128),
                                      E_absolute_position.dtype),
        mesh=mesh,
        scratch_types=scratch,
        compiler_params=pltpu.CompilerParams(use_tc_tiling_on_sc=False),
    )
    def gather_kernel(table_hbm, idx_hbm, out_hbm, idx_v, *scr):
        rows = scr[:_DEPTH]
        gsem = scr[_DEPTH:2 * _DEPTH]
        wsem = scr[2 * _DEPTH:]
        wid = lax.axis_index("s") * _NC + lax.axis_index("c")
        base = wid * per_w

        pltpu.sync_copy(idx_hbm.at[wid], idx_v)

        def gth(c, k):
            return pltpu.make_async_copy(
                table_hbm.at[idx_v.at[c]], rows[k], gsem[k])

        def wrt(c, k):
            return pltpu.make_async_copy(
                rows[k],
                out_hbm.at[pl.ds(base + c * _GW, _GW), pl.ds(0, D)],
                wsem[k])

        for k in range(_DEPTH):
            gth(k, k).start()

        @pl.loop(0, n_groups)
        def _(t):
            c0 = t * _DEPTH
            for k in range(_DEPTH):
                gth(c0 + k, k).wait()
                wrt(c0 + k, k).start()
            for k in range(_DEPTH):
                cn = lax.rem(c0 + k + _DEPTH, n_chunks)
                wrt(c0 + k, k).wait()
                gth(cn, k).start()

        for k in range(_DEPTH):
            gth(k, k).wait()

    out = gather_kernel(table_lin, idx)
    return out[:, :D].reshape(B, H, D)


# speedup vs baseline: 2.0441x; 1.1105x over previous
import jax
import jax.numpy as jnp
from jax import lax
from jax.experimental import pallas as pl
from jax.experimental.pallas import tpu as pltpu
from jax.experimental.pallas import tpu_sc as plsc

_NC = 2
_NS = 16
_NW = _NC * _NS
_GW = 128
_DEPTH = 4
_BC = 8192


def _repack_block(t_ref, o_ref):
    h = _BC // 2
    o_ref[:, :64] = t_ref[:, :h].T
    o_ref[:, 64:] = t_ref[:, h:].T


def _repack(table_t):
    D, N = table_t.shape
    n_blocks = (N + _BC - 1) // _BC
    return pl.pallas_call(
        _repack_block,
        grid=(n_blocks,),
        in_specs=[pl.BlockSpec((D, _BC), lambda i: (0, i))],
        out_specs=pl.BlockSpec((_BC // 2, 2 * D), lambda i: (i, 0)),
        out_shape=jax.ShapeDtypeStruct((n_blocks * (_BC // 2), 2 * D),
                                       table_t.dtype),
    )(table_t)


def kernel(x, E_absolute_position):
    B, H = x.shape
    N, D = E_absolute_position.shape
    num_indices = B * H
    per_w = num_indices // _NW
    n_chunks = per_w // _GW
    n_groups = n_chunks // _DEPTH
    assert per_w * _NW == num_indices
    assert n_groups * _DEPTH * _GW == per_w

    x32 = x.astype(jnp.int32)
    j = x32 & (_BC - 1)
    idx = 2 * x32 - (x32 - j) - jnp.where(j < _BC // 2, 0, _BC - 1)
    idx = idx.reshape(_NW, n_chunks, _GW)

    packed = _repack(E_absolute_position.T)
    table_lin = packed.reshape(packed.shape[0] * 2, D)

    mesh = plsc.VectorSubcoreMesh(core_axis_name="c", subcore_axis_name="s")

    scratch = (
        [pltpu.VMEM((n_chunks, _GW), jnp.int32)]
        + [pltpu.VMEM((_GW, D), jnp.float32) for _ in range(_DEPTH)]
        + [pltpu.SemaphoreType.DMA for _ in range(2 * _DEPTH)]
    )

    @pl.kernel(
        out_type=jax.ShapeDtypeStruct((num_indices, 128),
                                      E_absolute_position.dtype),
        mesh=mesh,
        scratch_types=scratch,
        compiler_params=pltpu.CompilerParams(use_tc_tiling_on_sc=False),
    )
    def gather_kernel(table_hbm, idx_hbm, out_hbm, idx_v, *scr):
        rows = scr[:_DEPTH]
        gsem = scr[_DEPTH:2 * _DEPTH]
        wsem = scr[2 * _DEPTH:]
        wid = lax.axis_index("s") * _NC + lax.axis_index("c")
        base = wid * per_w

        pltpu.sync_copy(idx_hbm.at[wid], idx_v)

        def gth(c, k):
            return pltpu.make_async_copy(
                table_hbm.at[idx_v.at[c]], rows[k], gsem[k])

        def wrt(c, k):
            return pltpu.make_async_copy(
                rows[k],
                out_hbm.at[pl.ds(base + c * _GW, _GW), pl.ds(0, D)],
                wsem[k])

        for k in range(_DEPTH):
            gth(k, k).start()

        @pl.loop(0, n_groups)
        def _(t):
            c0 = t * _DEPTH
            for k in range(_DEPTH):
                gth(c0 + k, k).wait()
                wrt(c0 + k, k).start()
            for k in range(_DEPTH):
                cn = lax.rem(c0 + k + _DEPTH, n_chunks)
                wrt(c0 + k, k).wait()
                gth(cn, k).start()

        for k in range(_DEPTH):
            gth(k, k).wait()

    out = gather_kernel(table_lin, idx)
    return out[:, :D].reshape(B, H, D)


# speedup vs baseline: 2.1571x; 1.0553x over previous
import jax
import jax.numpy as jnp
from jax import lax
from jax.experimental import pallas as pl
from jax.experimental.pallas import tpu as pltpu
from jax.experimental.pallas import tpu_sc as plsc

_NC = 2
_NS = 16
_NW = _NC * _NS
_GW = 128
_DEPTH = 4
_BC = 16384


def _repack_block(t_ref, o_ref):
    h = _BC // 2
    o_ref[:, :64] = t_ref[:, :h].T
    o_ref[:, 64:] = t_ref[:, h:].T


def _repack(table_t):
    D, N = table_t.shape
    n_blocks = (N + _BC - 1) // _BC
    return pl.pallas_call(
        _repack_block,
        grid=(n_blocks,),
        in_specs=[pl.BlockSpec((D, _BC), lambda i: (0, i))],
        out_specs=pl.BlockSpec((_BC // 2, 2 * D), lambda i: (i, 0)),
        out_shape=jax.ShapeDtypeStruct((n_blocks * (_BC // 2), 2 * D),
                                       table_t.dtype),
    )(table_t)


def kernel(x, E_absolute_position):
    B, H = x.shape
    N, D = E_absolute_position.shape
    num_indices = B * H
    per_w = num_indices // _NW
    n_chunks = per_w // _GW
    n_groups = n_chunks // _DEPTH
    assert per_w * _NW == num_indices
    assert n_groups * _DEPTH * _GW == per_w

    x32 = x.astype(jnp.int32)
    j = x32 & (_BC - 1)
    idx = 2 * x32 - (x32 - j) - jnp.where(j < _BC // 2, 0, _BC - 1)
    idx = idx.reshape(_NW, n_chunks, _GW)

    packed = _repack(E_absolute_position.T)
    table_lin = packed.reshape(packed.shape[0] * 2, D)

    mesh = plsc.VectorSubcoreMesh(core_axis_name="c", subcore_axis_name="s")

    scratch = (
        [pltpu.VMEM((n_chunks, _GW), jnp.int32)]
        + [pltpu.VMEM((_GW, D), jnp.float32) for _ in range(_DEPTH)]
        + [pltpu.SemaphoreType.DMA for _ in range(2 * _DEPTH)]
    )

    @pl.kernel(
        out_type=jax.ShapeDtypeStruct((num_indices, 128),
                                      E_absolute_position.dtype),
        mesh=mesh,
        scratch_types=scratch,
        compiler_params=pltpu.CompilerParams(use_tc_tiling_on_sc=False),
    )
    def gather_kernel(table_hbm, idx_hbm, out_hbm, idx_v, *scr):
        rows = scr[:_DEPTH]
        gsem = scr[_DEPTH:2 * _DEPTH]
        wsem = scr[2 * _DEPTH:]
        wid = lax.axis_index("s") * _NC + lax.axis_index("c")
        base = wid * per_w

        pltpu.sync_copy(idx_hbm.at[wid], idx_v)

        def gth(c, k):
            return pltpu.make_async_copy(
                table_hbm.at[idx_v.at[c]], rows[k], gsem[k])

        def wrt(c, k):
            return pltpu.make_async_copy(
                rows[k],
                out_hbm.at[pl.ds(base + c * _GW, _GW), pl.ds(0, D)],
                wsem[k])

        for k in range(_DEPTH):
            gth(k, k).start()

        @pl.loop(0, n_groups)
        def _(t):
            c0 = t * _DEPTH
            for k in range(_DEPTH):
                gth(c0 + k, k).wait()
                wrt(c0 + k, k).start()
            for k in range(_DEPTH):
                cn = lax.rem(c0 + k + _DEPTH, n_chunks)
                wrt(c0 + k, k).wait()
                gth(cn, k).start()

        for k in range(_DEPTH):
            gth(k, k).wait()

    out = gather_kernel(table_lin, idx)
    return out[:, :D].reshape(B, H, D)


# speedup vs baseline: 2.2139x; 1.0263x over previous
import jax
import jax.numpy as jnp
from jax import lax
from jax.experimental import pallas as pl
from jax.experimental.pallas import tpu as pltpu
from jax.experimental.pallas import tpu_sc as plsc

_NC = 2
_NS = 16
_NW = _NC * _NS
_GW = 128
_DEPTH = 4
_BC = 32768


def _repack_block(t_ref, o_ref):
    h = _BC // 2
    o_ref[:, :64] = t_ref[:, :h].T
    o_ref[:, 64:] = t_ref[:, h:].T


def _repack(table_t):
    D, N = table_t.shape
    n_blocks = (N + _BC - 1) // _BC
    return pl.pallas_call(
        _repack_block,
        grid=(n_blocks,),
        in_specs=[pl.BlockSpec((D, _BC), lambda i: (0, i))],
        out_specs=pl.BlockSpec((_BC // 2, 2 * D), lambda i: (i, 0)),
        out_shape=jax.ShapeDtypeStruct((n_blocks * (_BC // 2), 2 * D),
                                       table_t.dtype),
    )(table_t)


def kernel(x, E_absolute_position):
    B, H = x.shape
    N, D = E_absolute_position.shape
    num_indices = B * H
    per_w = num_indices // _NW
    n_chunks = per_w // _GW
    n_groups = n_chunks // _DEPTH
    assert per_w * _NW == num_indices
    assert n_groups * _DEPTH * _GW == per_w

    x32 = x.astype(jnp.int32)
    j = x32 & (_BC - 1)
    idx = 2 * x32 - (x32 - j) - jnp.where(j < _BC // 2, 0, _BC - 1)
    idx = idx.reshape(_NW, n_chunks, _GW)

    packed = _repack(E_absolute_position.T)
    table_lin = packed.reshape(packed.shape[0] * 2, D)

    mesh = plsc.VectorSubcoreMesh(core_axis_name="c", subcore_axis_name="s")

    scratch = (
        [pltpu.VMEM((n_chunks, _GW), jnp.int32)]
        + [pltpu.VMEM((_GW, D), jnp.float32) for _ in range(_DEPTH)]
        + [pltpu.SemaphoreType.DMA for _ in range(2 * _DEPTH)]
    )

    @pl.kernel(
        out_type=jax.ShapeDtypeStruct((num_indices, 128),
                                      E_absolute_position.dtype),
        mesh=mesh,
        scratch_types=scratch,
        compiler_params=pltpu.CompilerParams(use_tc_tiling_on_sc=False),
    )
    def gather_kernel(table_hbm, idx_hbm, out_hbm, idx_v, *scr):
        rows = scr[:_DEPTH]
        gsem = scr[_DEPTH:2 * _DEPTH]
        wsem = scr[2 * _DEPTH:]
        wid = lax.axis_index("s") * _NC + lax.axis_index("c")
        base = wid * per_w

        pltpu.sync_copy(idx_hbm.at[wid], idx_v)

        def gth(c, k):
            return pltpu.make_async_copy(
                table_hbm.at[idx_v.at[c]], rows[k], gsem[k])

        def wrt(c, k):
            return pltpu.make_async_copy(
                rows[k],
                out_hbm.at[pl.ds(base + c * _GW, _GW), pl.ds(0, D)],
                wsem[k])

        for k in range(_DEPTH):
            gth(k, k).start()

        @pl.loop(0, n_groups)
        def _(t):
            c0 = t * _DEPTH
            for k in range(_DEPTH):
                gth(c0 + k, k).wait()
                wrt(c0 + k, k).start()
            for k in range(_DEPTH):
                cn = lax.rem(c0 + k + _DEPTH, n_chunks)
                wrt(c0 + k, k).wait()
                gth(cn, k).start()

        for k in range(_DEPTH):
            gth(k, k).wait()

    out = gather_kernel(table_lin, idx)
    return out[:, :D].reshape(B, H, D)
